# baseline (speedup 1.0000x reference)
import jax
import jax.numpy as jnp
import numpy as np
from jax.experimental import pallas as pl
from jax.experimental.pallas import tpu as pltpu

IMB = 8
MROWS = IMB * 1024
PAD = 32
SP = 8
K1B = 32
K1R = K1B * 32


def _conv1_kernel(xblk, wbig, b1big, out, x1p, v0, v1, v2):
    f32, bf16 = jnp.float32, jnp.bfloat16
    x1p[SP:SP + K1R, :] = xblk[...]
    x1p[0:SP, :] = jnp.zeros((SP, 96), f32)
    x1p[SP + K1R:, :] = jnp.zeros((SP, 96), f32)
    rid = jax.lax.broadcasted_iota(jnp.int32, (K1R, 1), 0) % 32
    for j, v in ((0, v0), (1, v1), (2, v2)):
        oy = j - 1
        src = x1p[SP + oy:SP + oy + K1R, :]
        if oy == -1:
            src = src * (rid != 0).astype(f32)
        elif oy == 1:
            src = src * (rid != 31).astype(f32)
        v[...] = src.astype(bf16)
    acc = jnp.dot(v0[...], wbig[0], preferred_element_type=f32)
    acc = acc + jnp.dot(v1[...], wbig[1], preferred_element_type=f32)
    acc = acc + jnp.dot(v2[...], wbig[2], preferred_element_type=f32)
    out[...] = jax.nn.relu(acc + b1big[...])


def _conv2_kernel(xblk, xm, w2c, b2, fcw, fcb, mblk, out, x2p, trip):
    f32, bf16 = jnp.float32, jnp.bfloat16
    c2 = 64
    x2p[SP:SP + MROWS, :] = xblk[...]
    x2p[0:SP, :] = jnp.zeros((SP, c2), f32)
    x2p[SP + MROWS:, :] = jnp.zeros((SP, c2), f32)
    trip[0:PAD, :] = jnp.zeros((PAD, 3 * c2), bf16)
    trip[PAD + MROWS:, :] = jnp.zeros((PAD, 3 * c2), bf16)
    for k in range(3):
        ox = k - 1
        trip[PAD:PAD + MROWS, k * c2:(k + 1) * c2] = (
            x2p[SP + ox:SP + ox + MROWS, :] * xm[:, k:k + 1]).astype(bf16)
    rid = jax.lax.broadcasted_iota(jnp.int32, (MROWS, 1), 0)
    yr = (rid // 32) % 32
    acc2 = jnp.dot(trip[PAD:PAD + MROWS, :], w2c[1], preferred_element_type=f32)
    t = jnp.dot(trip[PAD - 32:PAD - 32 + MROWS, :], w2c[0],
                preferred_element_type=f32)
    acc2 = acc2 + t * (yr != 0).astype(f32)
    t = jnp.dot(trip[PAD + 32:PAD + 32 + MROWS, :], w2c[2],
                preferred_element_type=f32)
    acc2 = acc2 + t * (yr != 31).astype(f32)
    acc2 = jax.nn.relu(acc2 + b2[...])
    pooled = jnp.mean(acc2.reshape(IMB, 1024, 128), axis=1)
    feat = jnp.dot(pooled, fcw[...], preferred_element_type=f32) + fcb[...]
    out[...] = feat * mblk[...]


def _sage_kernel(xg, w1l, b1l, w1r, b1r, w2l, b2l, w2r, b2r, out):
    f32 = jnp.float32
    x = xg[...]
    xr = x.reshape(16, 32, 128)
    s = jnp.sum(xr, axis=1, keepdims=True)
    mean = ((s - xr) * (1.0 / 31.0)).reshape(512, 128)
    h = jax.nn.relu(jnp.dot(mean, w1l[...], preferred_element_type=f32) + b1l[...]
                    + jnp.dot(x, w1r[...], preferred_element_type=f32) + b1r[...])
    hr = h.reshape(16, 32, 128)
    s2 = jnp.sum(hr, axis=1, keepdims=True)
    mean2 = ((s2 - hr) * (1.0 / 31.0)).reshape(512, 128)
    out[...] = (jnp.dot(mean2, w2l[...], preferred_element_type=f32) + b2l[...]
                + jnp.dot(h, w2r[...], preferred_element_type=f32) + b2r[...])


def _toeplitz_w1(conv1_w):
    wt = jnp.transpose(conv1_w, (2, 3, 1, 0))
    ii = jnp.arange(96)
    jj = jnp.arange(2048)
    xin = ii[:, None] // 3
    c = ii[:, None] % 3
    xout = jj[None, :] // 64
    o = jj[None, :] % 64
    d = xin - xout + 1
    valid = ((d >= 0) & (d <= 2)).astype(jnp.float32)
    dcl = jnp.clip(d, 0, 2)
    return (wt[:, dcl, c, o] * valid).astype(jnp.bfloat16)


def _col_masks():
    xcol = np.arange(1024) % 32
    xm = np.stack([((xcol + ox) >= 0) & ((xcol + ox) < 32) for ox in (-1, 0, 1)],
                  axis=1).astype(np.float32)
    return jnp.asarray(np.tile(xm, (IMB, 1)))


def kernel(x, mask, conv1_w, conv1_b, conv2_w, conv2_b, fc_w, fc_b,
           s1_wl, s1_bl, s1_wr, s1_br, s2_wl, s2_bl, s2_wr, s2_br):
    batch, cars, c, h, w = x.shape
    n_img = batch * cars
    f32, bf16 = jnp.float32, jnp.bfloat16
    full = lambda a: pl.BlockSpec(a.shape, lambda i: (0,) * a.ndim)
    par = pltpu.CompilerParams(dimension_semantics=("parallel",))

    xr = jnp.transpose(x, (0, 1, 3, 4, 2)).reshape(n_img * h, w * c)
    wbig = _toeplitz_w1(conv1_w)
    b1big = jnp.tile(conv1_b, (w,)).reshape(1, w * 64)
    h1 = pl.pallas_call(
        _conv1_kernel,
        grid=(n_img // K1B,),
        in_specs=[pl.BlockSpec((K1R, w * c), lambda i: (i, 0)),
                  full(wbig), full(b1big)],
        out_specs=pl.BlockSpec((K1R, w * 64), lambda i: (i, 0)),
        out_shape=jax.ShapeDtypeStruct((n_img * h, w * 64), f32),
        scratch_shapes=[
            pltpu.VMEM((K1R + 2 * SP, 96), f32),
            pltpu.VMEM((K1R, 96), bf16),
            pltpu.VMEM((K1R, 96), bf16),
            pltpu.VMEM((K1R, 96), bf16),
        ],
        compiler_params=par,
    )(xr, wbig, b1big)
    h1 = h1.reshape(n_img * h * w, 64)

    xmask = _col_masks()
    w2c = jnp.transpose(conv2_w, (2, 3, 1, 0)).reshape(3, 192, 128).astype(bf16)
    b2 = conv2_b.reshape(1, 128)
    fcb = fc_b.reshape(1, 128)
    mflat = mask.reshape(n_img, 1)
    feats = pl.pallas_call(
        _conv2_kernel,
        grid=(n_img // IMB,),
        in_specs=[
            pl.BlockSpec((MROWS, 64), lambda i: (i, 0)),
            full(xmask), full(w2c), full(b2), full(fc_w), full(fcb),
            pl.BlockSpec((IMB, 1), lambda i: (i, 0)),
        ],
        out_specs=pl.BlockSpec((IMB, 128), lambda i: (i, 0)),
        out_shape=jax.ShapeDtypeStruct((n_img, 128), f32),
        scratch_shapes=[
            pltpu.VMEM((MROWS + 2 * SP, 64), f32),
            pltpu.VMEM((MROWS + 2 * PAD, 192), bf16),
        ],
        compiler_params=par,
    )(h1, xmask, w2c, b2, fc_w.T, fcb, mflat)

    sage_in = (feats, s1_wl.T, s1_bl.reshape(1, 128), s1_wr.T,
               s1_br.reshape(1, 128), s2_wl.T, s2_bl.reshape(1, 128),
               s2_wr.T, s2_br.reshape(1, 128))
    res = pl.pallas_call(
        _sage_kernel,
        grid=(1,),
        in_specs=[full(a) for a in sage_in],
        out_specs=pl.BlockSpec((n_img, 128), lambda i: (0, 0)),
        out_shape=jax.ShapeDtypeStruct((n_img, 128), f32),
        compiler_params=par,
    )(*sage_in)

    return res.reshape(batch, cars, 128)


# speedup vs baseline: 5.6398x; 5.6398x over previous
import jax
import jax.numpy as jnp
import numpy as np
from jax.experimental import pallas as pl
from jax.experimental.pallas import tpu as pltpu

IMB = 8
MROWS = IMB * 1024
PAD = 32
SP = 8
K1B = 32
K1R = K1B * 32


def _conv1_kernel(xblk, wbig, b1big, out, x1p, v0, v1, v2):
    f32, bf16 = jnp.float32, jnp.bfloat16
    x1p[SP:SP + K1R, :] = xblk[...]
    x1p[0:SP, :] = jnp.zeros((SP, 96), f32)
    x1p[SP + K1R:, :] = jnp.zeros((SP, 96), f32)
    rid = jax.lax.broadcasted_iota(jnp.int32, (K1R, 96), 0) & 31
    zero = jnp.zeros((K1R, 96), f32)
    for j, v in ((0, v0), (1, v1), (2, v2)):
        oy = j - 1
        src = x1p[SP + oy:SP + oy + K1R, :]
        if oy == -1:
            src = jnp.where(rid != 0, src, zero)
        elif oy == 1:
            src = jnp.where(rid != 31, src, zero)
        v[...] = src.astype(bf16)
    acc = jnp.dot(v0[...], wbig[0], preferred_element_type=f32)
    acc = acc + jnp.dot(v1[...], wbig[1], preferred_element_type=f32)
    acc = acc + jnp.dot(v2[...], wbig[2], preferred_element_type=f32)
    out[...] = jax.nn.relu(acc + b1big[...])


def _conv2_kernel(xblk, w2c, b2, fcw, fcb, mblk, out, x2p, trip):
    f32, bf16 = jnp.float32, jnp.bfloat16
    c2 = 64
    x2p[SP:SP + MROWS, :] = xblk[...]
    x2p[0:SP, :] = jnp.zeros((SP, c2), f32)
    x2p[SP + MROWS:, :] = jnp.zeros((SP, c2), f32)
    trip[0:PAD, :] = jnp.zeros((PAD, 3 * c2), bf16)
    trip[PAD + MROWS:, :] = jnp.zeros((PAD, 3 * c2), bf16)
    xc = jax.lax.broadcasted_iota(jnp.int32, (MROWS, c2), 0) & 31
    zero = jnp.zeros((MROWS, c2), f32)
    trip[PAD:PAD + MROWS, 0:c2] = jnp.where(
        xc != 0, x2p[SP - 1:SP - 1 + MROWS, :], zero).astype(bf16)
    trip[PAD:PAD + MROWS, c2:2 * c2] = x2p[SP:SP + MROWS, :].astype(bf16)
    trip[PAD:PAD + MROWS, 2 * c2:3 * c2] = jnp.where(
        xc != 31, x2p[SP + 1:SP + 1 + MROWS, :], zero).astype(bf16)
    yr = (jax.lax.broadcasted_iota(jnp.int32, (MROWS, 128), 0) >> 5) & 31
    zero2 = jnp.zeros((MROWS, 128), f32)
    acc2 = jnp.dot(trip[PAD:PAD + MROWS, :], w2c[1], preferred_element_type=f32)
    t = jnp.dot(trip[PAD - 32:PAD - 32 + MROWS, :], w2c[0],
                preferred_element_type=f32)
    acc2 = acc2 + jnp.where(yr != 0, t, zero2)
    t = jnp.dot(trip[PAD + 32:PAD + 32 + MROWS, :], w2c[2],
                preferred_element_type=f32)
    acc2 = acc2 + jnp.where(yr != 31, t, zero2)
    acc2 = jax.nn.relu(acc2 + b2[...])
    pooled = jnp.mean(acc2.reshape(IMB, 1024, 128), axis=1)
    feat = jnp.dot(pooled, fcw[...], preferred_element_type=f32) + fcb[...]
    out[...] = feat * mblk[...]


def _sage_kernel(xg, w1l, b1l, w1r, b1r, w2l, b2l, w2r, b2r, out):
    f32 = jnp.float32
    x = xg[...]
    xr = x.reshape(16, 32, 128)
    s = jnp.sum(xr, axis=1, keepdims=True)
    mean = ((s - xr) * (1.0 / 31.0)).reshape(512, 128)
    h = jax.nn.relu(jnp.dot(mean, w1l[...], preferred_element_type=f32) + b1l[...]
                    + jnp.dot(x, w1r[...], preferred_element_type=f32) + b1r[...])
    hr = h.reshape(16, 32, 128)
    s2 = jnp.sum(hr, axis=1, keepdims=True)
    mean2 = ((s2 - hr) * (1.0 / 31.0)).reshape(512, 128)
    out[...] = (jnp.dot(mean2, w2l[...], preferred_element_type=f32) + b2l[...]
                + jnp.dot(h, w2r[...], preferred_element_type=f32) + b2r[...])


def _toeplitz_w1(conv1_w):
    wt = jnp.transpose(conv1_w, (2, 3, 1, 0))
    acc = jnp.zeros((3, 32, 3, 32, 64), jnp.float32)
    for kx in range(3):
        se = np.zeros((32, 32), np.float32)
        for xout in range(32):
            xin = xout + kx - 1
            if 0 <= xin < 32:
                se[xin, xout] = 1.0
        se = jnp.asarray(se)
        acc = acc + (se[None, :, None, :, None]
                     * wt[:, kx][:, None, :, None, :])
    return acc.reshape(3, 96, 2048).astype(jnp.bfloat16)


def kernel(x, mask, conv1_w, conv1_b, conv2_w, conv2_b, fc_w, fc_b,
           s1_wl, s1_bl, s1_wr, s1_br, s2_wl, s2_bl, s2_wr, s2_br):
    batch, cars, c, h, w = x.shape
    n_img = batch * cars
    f32, bf16 = jnp.float32, jnp.bfloat16
    full = lambda a: pl.BlockSpec(a.shape, lambda i: (0,) * a.ndim)
    par = pltpu.CompilerParams(dimension_semantics=("parallel",))

    xr = jnp.transpose(x, (0, 1, 3, 4, 2)).reshape(n_img * h, w * c)
    wbig = _toeplitz_w1(conv1_w)
    b1big = jnp.tile(conv1_b, (w,)).reshape(1, w * 64)
    h1 = pl.pallas_call(
        _conv1_kernel,
        grid=(n_img // K1B,),
        in_specs=[pl.BlockSpec((K1R, w * c), lambda i: (i, 0)),
                  full(wbig), full(b1big)],
        out_specs=pl.BlockSpec((K1R, w * 64), lambda i: (i, 0)),
        out_shape=jax.ShapeDtypeStruct((n_img * h, w * 64), f32),
        scratch_shapes=[
            pltpu.VMEM((K1R + 2 * SP, 96), f32),
            pltpu.VMEM((K1R, 96), bf16),
            pltpu.VMEM((K1R, 96), bf16),
            pltpu.VMEM((K1R, 96), bf16),
        ],
        compiler_params=par,
    )(xr, wbig, b1big)
    h1 = h1.reshape(n_img * h * w, 64)

    w2c = jnp.transpose(conv2_w, (2, 3, 1, 0)).reshape(3, 192, 128).astype(bf16)
    b2 = conv2_b.reshape(1, 128)
    fcb = fc_b.reshape(1, 128)
    mflat = mask.reshape(n_img, 1)
    feats = pl.pallas_call(
        _conv2_kernel,
        grid=(n_img // IMB,),
        in_specs=[
            pl.BlockSpec((MROWS, 64), lambda i: (i, 0)),
            full(w2c), full(b2), full(fc_w), full(fcb),
            pl.BlockSpec((IMB, 1), lambda i: (i, 0)),
        ],
        out_specs=pl.BlockSpec((IMB, 128), lambda i: (i, 0)),
        out_shape=jax.ShapeDtypeStruct((n_img, 128), f32),
        scratch_shapes=[
            pltpu.VMEM((MROWS + 2 * SP, 64), f32),
            pltpu.VMEM((MROWS + 2 * PAD, 192), bf16),
        ],
        compiler_params=par,
    )(h1, w2c, b2, fc_w.T, fcb, mflat)

    sage_in = (feats, s1_wl.T, s1_bl.reshape(1, 128), s1_wr.T,
               s1_br.reshape(1, 128), s2_wl.T, s2_bl.reshape(1, 128),
               s2_wr.T, s2_br.reshape(1, 128))
    res = pl.pallas_call(
        _sage_kernel,
        grid=(1,),
        in_specs=[full(a) for a in sage_in],
        out_specs=pl.BlockSpec((n_img, 128), lambda i: (0, 0)),
        out_shape=jax.ShapeDtypeStruct((n_img, 128), f32),
        compiler_params=par,
    )(*sage_in)

    return res.reshape(batch, cars, 128)


# speedup vs baseline: 7.1938x; 1.2755x over previous
import jax
import jax.numpy as jnp
import numpy as np
from jax.experimental import pallas as pl
from jax.experimental.pallas import tpu as pltpu

IMB = 8
MROWS = IMB * 1024
PAD = 32
SP = 8
K1B = 32
K1R = K1B * 32


def _conv1_kernel(xblk, wbig, b1big, out, x1p, v0, v1, v2):
    f32, bf16 = jnp.float32, jnp.bfloat16
    for cc in range(3):
        x1p[SP:SP + K1R, cc * 32:(cc + 1) * 32] = (
            xblk[:, cc, :, :].reshape(K1R, 32))
    x1p[0:SP, :] = jnp.zeros((SP, 96), f32)
    x1p[SP + K1R:, :] = jnp.zeros((SP, 96), f32)
    rid = jax.lax.broadcasted_iota(jnp.int32, (K1R, 96), 0) & 31
    zero = jnp.zeros((K1R, 96), f32)
    for j, v in ((0, v0), (1, v1), (2, v2)):
        oy = j - 1
        src = x1p[SP + oy:SP + oy + K1R, :]
        if oy == -1:
            src = jnp.where(rid != 0, src, zero)
        elif oy == 1:
            src = jnp.where(rid != 31, src, zero)
        v[...] = src.astype(bf16)
    acc = jnp.dot(v0[...], wbig[0], preferred_element_type=f32)
    acc = acc + jnp.dot(v1[...], wbig[1], preferred_element_type=f32)
    acc = acc + jnp.dot(v2[...], wbig[2], preferred_element_type=f32)
    out[...] = jax.nn.relu(acc + b1big[...])


def _conv2_kernel(xblk, w2mid, w2pair, b2, fcw, fcb, mblk, out, x2p, trip):
    f32, bf16 = jnp.float32, jnp.bfloat16
    c2 = 64
    x2p[SP:SP + MROWS, :] = xblk[...]
    x2p[0:SP, :] = jnp.zeros((SP, c2), f32)
    x2p[SP + MROWS:, :] = jnp.zeros((SP, c2), f32)
    trip[0:PAD, :] = jnp.zeros((PAD, 3 * c2), bf16)
    trip[PAD + MROWS:, :] = jnp.zeros((PAD, 3 * c2), bf16)
    xc = jax.lax.broadcasted_iota(jnp.int32, (MROWS, c2), 0) & 31
    zero = jnp.zeros((MROWS, c2), f32)
    trip[PAD:PAD + MROWS, 0:c2] = jnp.where(
        xc != 0, x2p[SP - 1:SP - 1 + MROWS, :], zero).astype(bf16)
    trip[PAD:PAD + MROWS, c2:2 * c2] = x2p[SP:SP + MROWS, :].astype(bf16)
    trip[PAD:PAD + MROWS, 2 * c2:3 * c2] = jnp.where(
        xc != 31, x2p[SP + 1:SP + 1 + MROWS, :], zero).astype(bf16)
    yr = (jax.lax.broadcasted_iota(jnp.int32, (MROWS, 128), 0) >> 5) & 31
    zero2 = jnp.zeros((MROWS, 128), f32)
    zrow = jnp.zeros((32, 128), f32)
    acc2 = jnp.dot(trip[PAD:PAD + MROWS, :], w2mid[...],
                   preferred_element_type=f32)
    a = jnp.dot(trip[PAD:PAD + MROWS, :], w2pair[...],
                preferred_element_type=f32)
    up = jnp.concatenate([zrow, a[0:MROWS - 32, 0:128]], axis=0)
    dn = jnp.concatenate([a[32:MROWS, 128:256], zrow], axis=0)
    acc2 = acc2 + jnp.where(yr != 0, up, zero2)
    acc2 = acc2 + jnp.where(yr != 31, dn, zero2)
    acc2 = jax.nn.relu(acc2 + b2[...])
    pooled = jnp.mean(acc2.reshape(IMB, 1024, 128), axis=1)
    feat = jnp.dot(pooled, fcw[...], preferred_element_type=f32) + fcb[...]
    out[...] = feat * mblk[...]


def _sage_kernel(xg, w1l, b1l, w1r, b1r, w2l, b2l, w2r, b2r, out):
    f32 = jnp.float32
    x = xg[...]
    xr = x.reshape(16, 32, 128)
    s = jnp.sum(xr, axis=1, keepdims=True)
    mean = ((s - xr) * (1.0 / 31.0)).reshape(512, 128)
    h = jax.nn.relu(jnp.dot(mean, w1l[...], preferred_element_type=f32) + b1l[...]
                    + jnp.dot(x, w1r[...], preferred_element_type=f32) + b1r[...])
    hr = h.reshape(16, 32, 128)
    s2 = jnp.sum(hr, axis=1, keepdims=True)
    mean2 = ((s2 - hr) * (1.0 / 31.0)).reshape(512, 128)
    out[...] = (jnp.dot(mean2, w2l[...], preferred_element_type=f32) + b2l[...]
                + jnp.dot(h, w2r[...], preferred_element_type=f32) + b2r[...])


def _toeplitz_w1(conv1_w):
    wt = jnp.transpose(conv1_w, (2, 3, 1, 0))
    acc = jnp.zeros((3, 3, 32, 32, 64), jnp.float32)
    for kx in range(3):
        se = np.zeros((32, 32), np.float32)
        for xout in range(32):
            xin = xout + kx - 1
            if 0 <= xin < 32:
                se[xin, xout] = 1.0
        se = jnp.asarray(se)
        acc = acc + (se[None, None, :, :, None]
                     * wt[:, kx][:, :, None, None, :])
    return acc.reshape(3, 96, 2048).astype(jnp.bfloat16)


def kernel(x, mask, conv1_w, conv1_b, conv2_w, conv2_b, fc_w, fc_b,
           s1_wl, s1_bl, s1_wr, s1_br, s2_wl, s2_bl, s2_wr, s2_br):
    batch, cars, c, h, w = x.shape
    n_img = batch * cars
    f32, bf16 = jnp.float32, jnp.bfloat16
    full = lambda a: pl.BlockSpec(a.shape, lambda i: (0,) * a.ndim)
    par = pltpu.CompilerParams(dimension_semantics=("parallel",))

    xr = x.reshape(n_img, c, h, w)
    wbig = _toeplitz_w1(conv1_w)
    b1big = jnp.tile(conv1_b, (w,)).reshape(1, w * 64)
    h1 = pl.pallas_call(
        _conv1_kernel,
        grid=(n_img // K1B,),
        in_specs=[pl.BlockSpec((K1B, c, h, w), lambda i: (i, 0, 0, 0)),
                  full(wbig), full(b1big)],
        out_specs=pl.BlockSpec((K1R, w * 64), lambda i: (i, 0)),
        out_shape=jax.ShapeDtypeStruct((n_img * h, w * 64), f32),
        scratch_shapes=[
            pltpu.VMEM((K1R + 2 * SP, 96), f32),
            pltpu.VMEM((K1R, 96), bf16),
            pltpu.VMEM((K1R, 96), bf16),
            pltpu.VMEM((K1R, 96), bf16),
        ],
        compiler_params=par,
    )(xr, wbig, b1big)
    h1 = h1.reshape(n_img * h * w, 64)

    w2c = jnp.transpose(conv2_w, (2, 3, 1, 0)).reshape(3, 192, 128).astype(bf16)
    w2mid = w2c[1]
    w2pair = jnp.concatenate([w2c[0], w2c[2]], axis=1)
    b2 = conv2_b.reshape(1, 128)
    fcb = fc_b.reshape(1, 128)
    mflat = mask.reshape(n_img, 1)
    feats = pl.pallas_call(
        _conv2_kernel,
        grid=(n_img // IMB,),
        in_specs=[
            pl.BlockSpec((MROWS, 64), lambda i: (i, 0)),
            full(w2mid), full(w2pair), full(b2), full(fc_w), full(fcb),
            pl.BlockSpec((IMB, 1), lambda i: (i, 0)),
        ],
        out_specs=pl.BlockSpec((IMB, 128), lambda i: (i, 0)),
        out_shape=jax.ShapeDtypeStruct((n_img, 128), f32),
        scratch_shapes=[
            pltpu.VMEM((MROWS + 2 * SP, 64), f32),
            pltpu.VMEM((MROWS + 2 * PAD, 192), bf16),
        ],
        compiler_params=par,
    )(h1, w2mid, w2pair, b2, fc_w.T, fcb, mflat)

    sage_in = (feats, s1_wl.T, s1_bl.reshape(1, 128), s1_wr.T,
               s1_br.reshape(1, 128), s2_wl.T, s2_bl.reshape(1, 128),
               s2_wr.T, s2_br.reshape(1, 128))
    res = pl.pallas_call(
        _sage_kernel,
        grid=(1,),
        in_specs=[full(a) for a in sage_in],
        out_specs=pl.BlockSpec((n_img, 128), lambda i: (0, 0)),
        out_shape=jax.ShapeDtypeStruct((n_img, 128), f32),
        compiler_params=par,
    )(*sage_in)

    return res.reshape(batch, cars, 128)


# speedup vs baseline: 7.3207x; 1.0176x over previous
import jax
import jax.numpy as jnp
import numpy as np
from jax.experimental import pallas as pl
from jax.experimental.pallas import tpu as pltpu

IMB = 8
MROWS = IMB * 1024
PAD = 32
SP = 8
K1B = 32
K1R = K1B * 32


def _conv1_kernel(xblk, w01, w2, b1big, out, x1p, v01, v2):
    f32, bf16 = jnp.float32, jnp.bfloat16
    for cc in range(3):
        x1p[SP:SP + K1R, cc * 32:(cc + 1) * 32] = (
            xblk[:, cc, :, :].reshape(K1R, 32))
    x1p[0:SP, :] = jnp.zeros((SP, 96), f32)
    x1p[SP + K1R:, :] = jnp.zeros((SP, 96), f32)
    rid = jax.lax.broadcasted_iota(jnp.int32, (K1R, 96), 0) & 31
    zero = jnp.zeros((K1R, 96), f32)
    v01[:, 0:96] = jnp.where(
        rid != 0, x1p[SP - 1:SP - 1 + K1R, :], zero).astype(bf16)
    v01[:, 96:192] = x1p[SP:SP + K1R, :].astype(bf16)
    v2[...] = jnp.where(
        rid != 31, x1p[SP + 1:SP + 1 + K1R, :], zero).astype(bf16)
    acc = jnp.dot(v01[...], w01[...], preferred_element_type=f32)
    acc = acc + jnp.dot(v2[...], w2[...], preferred_element_type=f32)
    out[...] = jax.nn.relu(acc + b1big[...])


def _conv2_kernel(xblk, w2mid, w2pair, b2, fcw, fcb, mblk, out, x2p, trip):
    f32, bf16 = jnp.float32, jnp.bfloat16
    c2 = 64
    x2p[SP:SP + MROWS, :] = xblk[...]
    x2p[0:SP, :] = jnp.zeros((SP, c2), f32)
    x2p[SP + MROWS:, :] = jnp.zeros((SP, c2), f32)
    trip[0:PAD, :] = jnp.zeros((PAD, 3 * c2), bf16)
    trip[PAD + MROWS:, :] = jnp.zeros((PAD, 3 * c2), bf16)
    xc = jax.lax.broadcasted_iota(jnp.int32, (MROWS, c2), 0) & 31
    zero = jnp.zeros((MROWS, c2), f32)
    trip[PAD:PAD + MROWS, 0:c2] = jnp.where(
        xc != 0, x2p[SP - 1:SP - 1 + MROWS, :], zero).astype(bf16)
    trip[PAD:PAD + MROWS, c2:2 * c2] = x2p[SP:SP + MROWS, :].astype(bf16)
    trip[PAD:PAD + MROWS, 2 * c2:3 * c2] = jnp.where(
        xc != 31, x2p[SP + 1:SP + 1 + MROWS, :], zero).astype(bf16)
    yr = (jax.lax.broadcasted_iota(jnp.int32, (MROWS, 128), 0) >> 5) & 31
    zero2 = jnp.zeros((MROWS, 128), f32)
    zrow = jnp.zeros((32, 128), f32)
    acc2 = jnp.dot(trip[PAD:PAD + MROWS, :], w2mid[...],
                   preferred_element_type=f32)
    a = jnp.dot(trip[PAD:PAD + MROWS, :], w2pair[...],
                preferred_element_type=f32)
    up = jnp.concatenate([zrow, a[0:MROWS - 32, 0:128]], axis=0)
    dn = jnp.concatenate([a[32:MROWS, 128:256], zrow], axis=0)
    acc2 = acc2 + jnp.where(yr != 0, up, zero2)
    acc2 = acc2 + jnp.where(yr != 31, dn, zero2)
    acc2 = jax.nn.relu(acc2 + b2[...])
    pooled = jnp.mean(acc2.reshape(IMB, 1024, 128), axis=1)
    feat = jnp.dot(pooled, fcw[...], preferred_element_type=f32) + fcb[...]
    out[...] = feat * mblk[...]


def _sage_kernel(xg, w1l, b1l, w1r, b1r, w2l, b2l, w2r, b2r, out):
    f32 = jnp.float32
    x = xg[...]
    xr = x.reshape(16, 32, 128)
    s = jnp.sum(xr, axis=1, keepdims=True)
    mean = ((s - xr) * (1.0 / 31.0)).reshape(512, 128)
    h = jax.nn.relu(jnp.dot(mean, w1l[...], preferred_element_type=f32) + b1l[...]
                    + jnp.dot(x, w1r[...], preferred_element_type=f32) + b1r[...])
    hr = h.reshape(16, 32, 128)
    s2 = jnp.sum(hr, axis=1, keepdims=True)
    mean2 = ((s2 - hr) * (1.0 / 31.0)).reshape(512, 128)
    out[...] = (jnp.dot(mean2, w2l[...], preferred_element_type=f32) + b2l[...]
                + jnp.dot(h, w2r[...], preferred_element_type=f32) + b2r[...])


def _toeplitz_w1(conv1_w):
    wt = jnp.transpose(conv1_w, (2, 3, 1, 0))
    acc = jnp.zeros((3, 3, 32, 32, 64), jnp.float32)
    for kx in range(3):
        se = np.zeros((32, 32), np.float32)
        for xout in range(32):
            xin = xout + kx - 1
            if 0 <= xin < 32:
                se[xin, xout] = 1.0
        se = jnp.asarray(se)
        acc = acc + (se[None, None, :, :, None]
                     * wt[:, kx][:, :, None, None, :])
    return acc.reshape(3, 96, 2048).astype(jnp.bfloat16)


def kernel(x, mask, conv1_w, conv1_b, conv2_w, conv2_b, fc_w, fc_b,
           s1_wl, s1_bl, s1_wr, s1_br, s2_wl, s2_bl, s2_wr, s2_br):
    batch, cars, c, h, w = x.shape
    n_img = batch * cars
    f32, bf16 = jnp.float32, jnp.bfloat16
    full = lambda a: pl.BlockSpec(a.shape, lambda i: (0,) * a.ndim)
    par = pltpu.CompilerParams(dimension_semantics=("parallel",))

    xr = x.reshape(n_img, c, h, w)
    wbig = _toeplitz_w1(conv1_w)
    w01 = jnp.concatenate([wbig[0], wbig[1]], axis=0)
    b1big = jnp.tile(conv1_b, (w,)).reshape(1, w * 64)
    h1 = pl.pallas_call(
        _conv1_kernel,
        grid=(n_img // K1B,),
        in_specs=[pl.BlockSpec((K1B, c, h, w), lambda i: (i, 0, 0, 0)),
                  full(w01), full(wbig[2]), full(b1big)],
        out_specs=pl.BlockSpec((K1R, w * 64), lambda i: (i, 0)),
        out_shape=jax.ShapeDtypeStruct((n_img * h, w * 64), f32),
        scratch_shapes=[
            pltpu.VMEM((K1R + 2 * SP, 96), f32),
            pltpu.VMEM((K1R, 192), bf16),
            pltpu.VMEM((K1R, 96), bf16),
        ],
        compiler_params=par,
    )(xr, w01, wbig[2], b1big)
    h1 = h1.reshape(n_img * h * w, 64)

    w2c = jnp.transpose(conv2_w, (2, 3, 1, 0)).reshape(3, 192, 128).astype(bf16)
    w2mid = w2c[1]
    w2pair = jnp.concatenate([w2c[0], w2c[2]], axis=1)
    b2 = conv2_b.reshape(1, 128)
    fcb = fc_b.reshape(1, 128)
    mflat = mask.reshape(n_img, 1)
    feats = pl.pallas_call(
        _conv2_kernel,
        grid=(n_img // IMB,),
        in_specs=[
            pl.BlockSpec((MROWS, 64), lambda i: (i, 0)),
            full(w2mid), full(w2pair), full(b2), full(fc_w), full(fcb),
            pl.BlockSpec((IMB, 1), lambda i: (i, 0)),
        ],
        out_specs=pl.BlockSpec((IMB, 128), lambda i: (i, 0)),
        out_shape=jax.ShapeDtypeStruct((n_img, 128), f32),
        scratch_shapes=[
            pltpu.VMEM((MROWS + 2 * SP, 64), f32),
            pltpu.VMEM((MROWS + 2 * PAD, 192), bf16),
        ],
        compiler_params=par,
    )(h1, w2mid, w2pair, b2, fc_w.T, fcb, mflat)

    sage_in = (feats, s1_wl.T, s1_bl.reshape(1, 128), s1_wr.T,
               s1_br.reshape(1, 128), s2_wl.T, s2_bl.reshape(1, 128),
               s2_wr.T, s2_br.reshape(1, 128))
    res = pl.pallas_call(
        _sage_kernel,
        grid=(1,),
        in_specs=[full(a) for a in sage_in],
        out_specs=pl.BlockSpec((n_img, 128), lambda i: (0, 0)),
        out_shape=jax.ShapeDtypeStruct((n_img, 128), f32),
        compiler_params=par,
    )(*sage_in)

    return res.reshape(batch, cars, 128)


# speedup vs baseline: 14.1831x; 1.9374x over previous
import jax
import jax.numpy as jnp
import numpy as np
from jax.experimental import pallas as pl
from jax.experimental.pallas import tpu as pltpu

IMB = 8
K2R = IMB * 32
SP = 8
K1B = 32
K1R = K1B * 32
VW = 64 + 2048 + 64


def _conv1_kernel(xblk, w01, w2, b1big, out, x1p, v01, v2):
    f32, bf16 = jnp.float32, jnp.bfloat16
    for cc in range(3):
        x1p[SP:SP + K1R, cc * 32:(cc + 1) * 32] = (
            xblk[:, cc, :, :].reshape(K1R, 32))
    x1p[0:SP, :] = jnp.zeros((SP, 96), f32)
    x1p[SP + K1R:, :] = jnp.zeros((SP, 96), f32)
    rid = jax.lax.broadcasted_iota(jnp.int32, (K1R, 96), 0) & 31
    zero = jnp.zeros((K1R, 96), f32)
    v01[:, 0:96] = jnp.where(
        rid != 0, x1p[SP - 1:SP - 1 + K1R, :], zero).astype(bf16)
    v01[:, 96:192] = x1p[SP:SP + K1R, :].astype(bf16)
    v2[...] = jnp.where(
        rid != 31, x1p[SP + 1:SP + 1 + K1R, :], zero).astype(bf16)
    acc = jnp.dot(v01[...], w01[...], preferred_element_type=f32)
    acc = acc + jnp.dot(v2[...], w2[...], preferred_element_type=f32)
    out[...] = jax.nn.relu(acc + b1big[...])


def _conv2_kernel(xblk, wg, b2, fcw, fcb, mblk, out, x2w, v0, v1, v2):
    f32, bf16 = jnp.float32, jnp.bfloat16
    x2w[SP:SP + K2R, :] = xblk[...]
    x2w[0:SP, :] = jnp.zeros((SP, 2048), f32)
    x2w[SP + K2R:, :] = jnp.zeros((SP, 2048), f32)
    rid = jax.lax.broadcasted_iota(jnp.int32, (K2R, 2048), 0) & 31
    zero = jnp.zeros((K2R, 2048), f32)
    zpad = jnp.zeros((K2R, 64), bf16)
    for v in (v0, v1, v2):
        v[:, 0:64] = zpad
        v[:, 64 + 2048:] = zpad
    v0[:, 64:64 + 2048] = jnp.where(
        rid != 0, x2w[SP - 1:SP - 1 + K2R, :], zero).astype(bf16)
    v1[:, 64:64 + 2048] = x2w[SP:SP + K2R, :].astype(bf16)
    v2[:, 64:64 + 2048] = jnp.where(
        rid != 31, x2w[SP + 1:SP + 1 + K2R, :], zero).astype(bf16)
    pp = jnp.zeros((K2R, 256), f32)
    for g in range(16):
        lo = g * 128
        a = jnp.dot(v0[:, lo:lo + 256], wg[0], preferred_element_type=f32)
        a = a + jnp.dot(v1[:, lo:lo + 256], wg[1], preferred_element_type=f32)
        a = a + jnp.dot(v2[:, lo:lo + 256], wg[2], preferred_element_type=f32)
        pp = pp + jax.nn.relu(a + b2[...])
    pooled = jnp.sum(pp.reshape(IMB, 32, 2, 128), axis=(1, 2)) * (1.0 / 1024.0)
    feat = jnp.dot(pooled, fcw[...], preferred_element_type=f32) + fcb[...]
    out[...] = feat * mblk[...]


def _sage_kernel(xg, w1l, b1l, w1r, b1r, w2l, b2l, w2r, b2r, out):
    f32 = jnp.float32
    x = xg[...]
    xr = x.reshape(16, 32, 128)
    s = jnp.sum(xr, axis=1, keepdims=True)
    mean = ((s - xr) * (1.0 / 31.0)).reshape(512, 128)
    h = jax.nn.relu(jnp.dot(mean, w1l[...], preferred_element_type=f32) + b1l[...]
                    + jnp.dot(x, w1r[...], preferred_element_type=f32) + b1r[...])
    hr = h.reshape(16, 32, 128)
    s2 = jnp.sum(hr, axis=1, keepdims=True)
    mean2 = ((s2 - hr) * (1.0 / 31.0)).reshape(512, 128)
    out[...] = (jnp.dot(mean2, w2l[...], preferred_element_type=f32) + b2l[...]
                + jnp.dot(h, w2r[...], preferred_element_type=f32) + b2r[...])


def _toeplitz_w1(conv1_w):
    wt = jnp.transpose(conv1_w, (2, 3, 1, 0))
    acc = jnp.zeros((3, 3, 32, 32, 64), jnp.float32)
    for kx in range(3):
        se = np.zeros((32, 32), np.float32)
        for xout in range(32):
            xin = xout + kx - 1
            if 0 <= xin < 32:
                se[xin, xout] = 1.0
        se = jnp.asarray(se)
        acc = acc + (se[None, None, :, :, None]
                     * wt[:, kx][:, :, None, None, :])
    return acc.reshape(3, 96, 2048).astype(jnp.bfloat16)


def _window_w2(conv2_w):
    wt = jnp.transpose(conv2_w, (2, 3, 1, 0))
    wg = jnp.zeros((3, 256, 256), jnp.float32)
    for xl in range(4):
        for xo in range(2):
            kx = xl - xo
            if 0 <= kx <= 2:
                wg = wg.at[:, xl * 64:(xl + 1) * 64,
                           xo * 128:(xo + 1) * 128].set(wt[:, kx])
    return wg.astype(jnp.bfloat16)


def kernel(x, mask, conv1_w, conv1_b, conv2_w, conv2_b, fc_w, fc_b,
           s1_wl, s1_bl, s1_wr, s1_br, s2_wl, s2_bl, s2_wr, s2_br):
    batch, cars, c, h, w = x.shape
    n_img = batch * cars
    f32, bf16 = jnp.float32, jnp.bfloat16
    full = lambda a: pl.BlockSpec(a.shape, lambda i: (0,) * a.ndim)
    par = pltpu.CompilerParams(dimension_semantics=("parallel",))

    xr = x.reshape(n_img, c, h, w)
    wbig = _toeplitz_w1(conv1_w)
    w01 = jnp.concatenate([wbig[0], wbig[1]], axis=0)
    b1big = jnp.tile(conv1_b, (w,)).reshape(1, w * 64)
    h1 = pl.pallas_call(
        _conv1_kernel,
        grid=(n_img // K1B,),
        in_specs=[pl.BlockSpec((K1B, c, h, w), lambda i: (i, 0, 0, 0)),
                  full(w01), full(wbig[2]), full(b1big)],
        out_specs=pl.BlockSpec((K1R, w * 64), lambda i: (i, 0)),
        out_shape=jax.ShapeDtypeStruct((n_img * h, w * 64), f32),
        scratch_shapes=[
            pltpu.VMEM((K1R + 2 * SP, 96), f32),
            pltpu.VMEM((K1R, 192), bf16),
            pltpu.VMEM((K1R, 96), bf16),
        ],
        compiler_params=par,
    )(xr, w01, wbig[2], b1big)

    wg = _window_w2(conv2_w)
    b2w = jnp.tile(conv2_b, (2,)).reshape(1, 256)
    fcb = fc_b.reshape(1, 128)
    mflat = mask.reshape(n_img, 1)
    feats = pl.pallas_call(
        _conv2_kernel,
        grid=(n_img // IMB,),
        in_specs=[
            pl.BlockSpec((K2R, w * 64), lambda i: (i, 0)),
            full(wg), full(b2w), full(fc_w), full(fcb),
            pl.BlockSpec((IMB, 1), lambda i: (i, 0)),
        ],
        out_specs=pl.BlockSpec((IMB, 128), lambda i: (i, 0)),
        out_shape=jax.ShapeDtypeStruct((n_img, 128), f32),
        scratch_shapes=[
            pltpu.VMEM((K2R + 2 * SP, 2048), f32),
            pltpu.VMEM((K2R, VW), bf16),
            pltpu.VMEM((K2R, VW), bf16),
            pltpu.VMEM((K2R, VW), bf16),
        ],
        compiler_params=par,
    )(h1, wg, b2w, fc_w.T, fcb, mflat)

    sage_in = (feats, s1_wl.T, s1_bl.reshape(1, 128), s1_wr.T,
               s1_br.reshape(1, 128), s2_wl.T, s2_bl.reshape(1, 128),
               s2_wr.T, s2_br.reshape(1, 128))
    res = pl.pallas_call(
        _sage_kernel,
        grid=(1,),
        in_specs=[full(a) for a in sage_in],
        out_specs=pl.BlockSpec((n_img, 128), lambda i: (0, 0)),
        out_shape=jax.ShapeDtypeStruct((n_img, 128), f32),
        compiler_params=par,
    )(*sage_in)

    return res.reshape(batch, cars, 128)


# speedup vs baseline: 14.8131x; 1.0444x over previous
import jax
import jax.numpy as jnp
import numpy as np
from jax.experimental import pallas as pl
from jax.experimental.pallas import tpu as pltpu

IMB = 8
K2R = IMB * 32
SP = 8
K1B = 32
K1R = K1B * 32
VW = 64 + 2048 + 64


def _conv1_kernel(xblk, w01, w2, b1big, out, x1p, v01, v2):
    f32, bf16 = jnp.float32, jnp.bfloat16
    for cc in range(3):
        x1p[SP:SP + K1R, cc * 32:(cc + 1) * 32] = (
            xblk[:, cc, :, :].reshape(K1R, 32))
    x1p[0:SP, :] = jnp.zeros((SP, 96), f32)
    x1p[SP + K1R:, :] = jnp.zeros((SP, 96), f32)
    rid = jax.lax.broadcasted_iota(jnp.int32, (K1R, 96), 0) & 31
    zero = jnp.zeros((K1R, 96), f32)
    v01[:, 0:96] = jnp.where(
        rid != 0, x1p[SP - 1:SP - 1 + K1R, :], zero).astype(bf16)
    v01[:, 96:192] = x1p[SP:SP + K1R, :].astype(bf16)
    v2[...] = jnp.where(
        rid != 31, x1p[SP + 1:SP + 1 + K1R, :], zero).astype(bf16)
    acc = jnp.dot(v01[...], w01[...], preferred_element_type=f32)
    acc = acc + jnp.dot(v2[...], w2[...], preferred_element_type=f32)
    out[...] = jax.nn.relu(acc + b1big[...]).astype(bf16)


def _conv2_kernel(xblk, wg, b2, fcw, fcb, mblk, out, x2w, v0, v1, v2):
    f32, bf16 = jnp.float32, jnp.bfloat16
    x2w[SP:SP + K2R, :] = xblk[...].astype(f32)
    x2w[0:SP, :] = jnp.zeros((SP, 2048), f32)
    x2w[SP + K2R:, :] = jnp.zeros((SP, 2048), f32)
    rid = jax.lax.broadcasted_iota(jnp.int32, (K2R, 2048), 0) & 31
    zero = jnp.zeros((K2R, 2048), f32)
    zpad = jnp.zeros((K2R, 64), bf16)
    for v in (v0, v1, v2):
        v[:, 0:64] = zpad
        v[:, 64 + 2048:] = zpad
    v0[:, 64:64 + 2048] = jnp.where(
        rid != 0, x2w[SP - 1:SP - 1 + K2R, :], zero).astype(bf16)
    v1[:, 64:64 + 2048] = x2w[SP:SP + K2R, :].astype(bf16)
    v2[:, 64:64 + 2048] = jnp.where(
        rid != 31, x2w[SP + 1:SP + 1 + K2R, :], zero).astype(bf16)
    pp = jnp.zeros((K2R, 256), f32)
    for g in range(16):
        lo = g * 128
        a = jnp.dot(v0[:, lo:lo + 256], wg[0], preferred_element_type=f32)
        a = a + jnp.dot(v1[:, lo:lo + 256], wg[1], preferred_element_type=f32)
        a = a + jnp.dot(v2[:, lo:lo + 256], wg[2], preferred_element_type=f32)
        pp = pp + jax.nn.relu(a + b2[...])
    pooled = jnp.sum(pp.reshape(IMB, 32, 2, 128), axis=(1, 2)) * (1.0 / 1024.0)
    feat = jnp.dot(pooled, fcw[...], preferred_element_type=f32) + fcb[...]
    out[...] = feat * mblk[...]


def _sage_kernel(xg, w1l, b1l, w1r, b1r, w2l, b2l, w2r, b2r, out):
    f32 = jnp.float32
    x = xg[...]
    xr = x.reshape(16, 32, 128)
    s = jnp.sum(xr, axis=1, keepdims=True)
    mean = ((s - xr) * (1.0 / 31.0)).reshape(512, 128)
    h = jax.nn.relu(jnp.dot(mean, w1l[...], preferred_element_type=f32) + b1l[...]
                    + jnp.dot(x, w1r[...], preferred_element_type=f32) + b1r[...])
    hr = h.reshape(16, 32, 128)
    s2 = jnp.sum(hr, axis=1, keepdims=True)
    mean2 = ((s2 - hr) * (1.0 / 31.0)).reshape(512, 128)
    out[...] = (jnp.dot(mean2, w2l[...], preferred_element_type=f32) + b2l[...]
                + jnp.dot(h, w2r[...], preferred_element_type=f32) + b2r[...])


def _toeplitz_w1(conv1_w):
    wt = jnp.transpose(conv1_w, (2, 3, 1, 0))
    acc = jnp.zeros((3, 3, 32, 32, 64), jnp.float32)
    for kx in range(3):
        se = np.zeros((32, 32), np.float32)
        for xout in range(32):
            xin = xout + kx - 1
            if 0 <= xin < 32:
                se[xin, xout] = 1.0
        se = jnp.asarray(se)
        acc = acc + (se[None, None, :, :, None]
                     * wt[:, kx][:, :, None, None, :])
    return acc.reshape(3, 96, 2048).astype(jnp.bfloat16)


def _window_w2(conv2_w):
    wt = jnp.transpose(conv2_w, (2, 3, 1, 0))
    wg = jnp.zeros((3, 256, 256), jnp.float32)
    for xl in range(4):
        for xo in range(2):
            kx = xl - xo
            if 0 <= kx <= 2:
                wg = wg.at[:, xl * 64:(xl + 1) * 64,
                           xo * 128:(xo + 1) * 128].set(wt[:, kx])
    return wg.astype(jnp.bfloat16)


def kernel(x, mask, conv1_w, conv1_b, conv2_w, conv2_b, fc_w, fc_b,
           s1_wl, s1_bl, s1_wr, s1_br, s2_wl, s2_bl, s2_wr, s2_br):
    batch, cars, c, h, w = x.shape
    n_img = batch * cars
    f32, bf16 = jnp.float32, jnp.bfloat16
    full = lambda a: pl.BlockSpec(a.shape, lambda i: (0,) * a.ndim)
    par = pltpu.CompilerParams(dimension_semantics=("parallel",))

    xr = x.reshape(n_img, c, h, w)
    wbig = _toeplitz_w1(conv1_w)
    w01 = jnp.concatenate([wbig[0], wbig[1]], axis=0)
    b1big = jnp.tile(conv1_b, (w,)).reshape(1, w * 64)
    h1 = pl.pallas_call(
        _conv1_kernel,
        grid=(n_img // K1B,),
        in_specs=[pl.BlockSpec((K1B, c, h, w), lambda i: (i, 0, 0, 0)),
                  full(w01), full(wbig[2]), full(b1big)],
        out_specs=pl.BlockSpec((K1R, w * 64), lambda i: (i, 0)),
        out_shape=jax.ShapeDtypeStruct((n_img * h, w * 64), bf16),
        scratch_shapes=[
            pltpu.VMEM((K1R + 2 * SP, 96), f32),
            pltpu.VMEM((K1R, 192), bf16),
            pltpu.VMEM((K1R, 96), bf16),
        ],
        compiler_params=par,
    )(xr, w01, wbig[2], b1big)

    wg = _window_w2(conv2_w)
    b2w = jnp.tile(conv2_b, (2,)).reshape(1, 256)
    fcb = fc_b.reshape(1, 128)
    mflat = mask.reshape(n_img, 1)
    feats = pl.pallas_call(
        _conv2_kernel,
        grid=(n_img // IMB,),
        in_specs=[
            pl.BlockSpec((K2R, w * 64), lambda i: (i, 0)),
            full(wg), full(b2w), full(fc_w), full(fcb),
            pl.BlockSpec((IMB, 1), lambda i: (i, 0)),
        ],
        out_specs=pl.BlockSpec((IMB, 128), lambda i: (i, 0)),
        out_shape=jax.ShapeDtypeStruct((n_img, 128), f32),
        scratch_shapes=[
            pltpu.VMEM((K2R + 2 * SP, 2048), f32),
            pltpu.VMEM((K2R, VW), bf16),
            pltpu.VMEM((K2R, VW), bf16),
            pltpu.VMEM((K2R, VW), bf16),
        ],
        compiler_params=par,
    )(h1, wg, b2w, fc_w.T, fcb, mflat)

    sage_in = (feats, s1_wl.T, s1_bl.reshape(1, 128), s1_wr.T,
               s1_br.reshape(1, 128), s2_wl.T, s2_bl.reshape(1, 128),
               s2_wr.T, s2_br.reshape(1, 128))
    res = pl.pallas_call(
        _sage_kernel,
        grid=(1,),
        in_specs=[full(a) for a in sage_in],
        out_specs=pl.BlockSpec((n_img, 128), lambda i: (0, 0)),
        out_shape=jax.ShapeDtypeStruct((n_img, 128), f32),
        compiler_params=par,
    )(*sage_in)

    return res.reshape(batch, cars, 128)
